# baseline (speedup 1.0000x reference)
import functools

import jax
import jax.numpy as jnp
from jax import lax
from jax.experimental import pallas as pl
from jax.experimental.pallas import tpu as pltpu
from jax.experimental.pallas import tpu_sc as plsc

N_NODES = 10000
N_EDGES = 320000
L = 16
NS = 16
N_PAD = 10240
SLICE = N_PAD // NS
VPS = SLICE // L
E_PER_T = N_EDGES // NS

def _newton_rsqrt(x):
    i = lax.bitcast_convert_type(x, jnp.int32)
    i = jnp.int32(0x5F3759DF) - lax.shift_right_logical(i, 1)
    y = lax.bitcast_convert_type(i, jnp.float32)
    for _ in range(3):
        y = y * (1.5 - 0.5 * x * y * y)
    return y


def _sc_fused_body(src_hbm, dst_hbm, c_hbm, src_v, dst_v, hist_v, dinv_v,
                   red_v, sl_v, shared, shared2, sem, sem_d, sem_b):
    cid = lax.axis_index("c")
    sid = lax.axis_index("s")

    @pl.when(cid == 0)
    def _():
        ebase = sid * E_PER_T
        nbase = sid * SLICE
        cs = pltpu.async_copy(src_hbm.at[pl.ds(ebase, E_PER_T)], src_v, sem)
        cd = pltpu.async_copy(dst_hbm.at[pl.ds(ebase, E_PER_T)], dst_v, sem_d)

        zeros = jnp.zeros((L,), jnp.float32)
        ones = jnp.ones((L,), jnp.float32)

        @plsc.parallel_loop(0, N_PAD // L, unroll=8)
        def _(i):
            hist_v[pl.ds(i * L, L)] = zeros

        cd.wait()

        @plsc.parallel_loop(0, E_PER_T // L, unroll=5)
        def _(i):
            idx = dst_v[pl.ds(i * L, L)]
            plsc.addupdate_scatter(hist_v, [idx], ones)

        pltpu.sync_copy(hist_v, shared.at[sid])
        plsc.subcore_barrier()

        for k in range(NS):
            pltpu.sync_copy(shared.at[k, pl.ds(nbase, SLICE)], red_v.at[k])

        def _deg_red(j, carry):
            acc = red_v[0, pl.ds(j * L, L)]
            for k in range(1, NS):
                acc = acc + red_v[k, pl.ds(j * L, L)]
            sl_v[pl.ds(j * L, L)] = _newton_rsqrt(acc + 1.0)
            return carry

        lax.fori_loop(0, VPS, _deg_red, 0)

        pltpu.sync_copy(sl_v, shared2.at[pl.ds(nbase, SLICE)])
        plsc.subcore_barrier()

        cb = pltpu.async_copy(shared2, dinv_v, sem_b)

        @plsc.parallel_loop(0, N_PAD // L, unroll=8)
        def _(i):
            hist_v[pl.ds(i * L, L)] = zeros

        cb.wait()
        cs.wait()

        @plsc.parallel_loop(0, E_PER_T // L, unroll=5)
        def _(i):
            si = src_v[pl.ds(i * L, L)]
            di = dst_v[pl.ds(i * L, L)]
            dvals = plsc.load_gather(dinv_v, [di])
            plsc.addupdate_scatter(hist_v, [si], dvals)

        pltpu.sync_copy(hist_v, shared.at[sid])
        plsc.subcore_barrier()

        for k in range(NS):
            pltpu.sync_copy(shared.at[k, pl.ds(nbase, SLICE)], red_v.at[k])

        def _t_red(j, carry):
            acc = red_v[0, pl.ds(j * L, L)]
            for k in range(1, NS):
                acc = acc + red_v[k, pl.ds(j * L, L)]
            dv = sl_v[pl.ds(j * L, L)]
            sl_v[pl.ds(j * L, L)] = dv * (dv + acc)
            return carry

        lax.fori_loop(0, VPS, _t_red, 0)

        pltpu.sync_copy(sl_v, c_hbm.at[pl.ds(nbase, SLICE)])


@functools.cache
def _build_sc_kernel():
    mesh = plsc.VectorSubcoreMesh(core_axis_name="c", subcore_axis_name="s")
    params = pltpu.CompilerParams(needs_layout_passes=False)
    return pl.kernel(
        _sc_fused_body,
        mesh=mesh,
        out_type=jax.ShapeDtypeStruct((N_PAD,), jnp.float32),
        scratch_types=[
            pltpu.VMEM((E_PER_T,), jnp.int32),
            pltpu.VMEM((E_PER_T,), jnp.int32),
            pltpu.VMEM((N_PAD,), jnp.float32),
            pltpu.VMEM((N_PAD,), jnp.float32),
            pltpu.VMEM((NS, SLICE), jnp.float32),
            pltpu.VMEM((SLICE,), jnp.float32),
            pltpu.VMEM_SHARED((NS, N_PAD), jnp.float32),
            pltpu.VMEM_SHARED((N_PAD,), jnp.float32),
            pltpu.SemaphoreType.DMA,
            pltpu.SemaphoreType.DMA,
            pltpu.SemaphoreType.DMA,
        ],
        compiler_params=params,
    )


def _tc_final(c_ref, x_ref, w_ref, b_ref, wd_ref, bd_ref, out_ref):
    c = c_ref[:, :N_NODES]
    cx = lax.dot_general(c, x_ref[...], (((1,), (0,)), ((), ())),
                         preferred_element_type=jnp.float32)
    pooled = lax.dot_general(cx, w_ref[...], (((1,), (0,)), ((), ())),
                             preferred_element_type=jnp.float32)
    pooled = pooled + float(N_NODES) * b_ref[...]
    logits = lax.dot_general(pooled, wd_ref[...], (((1,), (0,)), ((), ())),
                             preferred_element_type=jnp.float32)
    logits = logits + bd_ref[...]
    m = jnp.max(logits, axis=1, keepdims=True)
    e = jnp.exp(logits - m)
    out_ref[...] = e / jnp.sum(e, axis=1, keepdims=True)


def kernel(x, edge_index, W, b, Wd, bd):
    sc_fused = _build_sc_kernel()
    src = edge_index[0].astype(jnp.int32)
    dst = edge_index[1].astype(jnp.int32)

    c = sc_fused(src, dst)

    out = pl.pallas_call(
        _tc_final,
        out_shape=jax.ShapeDtypeStruct((1, 10), jnp.float32),
    )(c.reshape(1, N_PAD), x, W, b.reshape(1, -1), Wd, bd.reshape(1, -1))
    return out.reshape(10)


# speedup vs baseline: 1.2380x; 1.2380x over previous
import functools

import jax
import jax.numpy as jnp
from jax import lax
from jax.experimental import pallas as pl
from jax.experimental.pallas import tpu as pltpu
from jax.experimental.pallas import tpu_sc as plsc

N_NODES = 10000
N_EDGES = 320000
L = 16
NS = 16
N_PAD = 10240
SLICE = N_PAD // NS
VPS = SLICE // L
E_PER_T = N_EDGES // NS

def _newton_rsqrt(x):
    i = lax.bitcast_convert_type(x, jnp.int32)
    i = jnp.int32(0x5F3759DF) - lax.shift_right_logical(i, 1)
    y = lax.bitcast_convert_type(i, jnp.float32)
    for _ in range(3):
        y = y * (1.5 - 0.5 * x * y * y)
    return y


def _sc_fused_body(edge_hbm, c_hbm, src_v, dst_v, hist_v, dinv_v,
                   red_v, sl_v, shared, shared2, sem, sem_d, sem_b):
    cid = lax.axis_index("c")
    sid = lax.axis_index("s")

    @pl.when(cid == 0)
    def _():
        ebase = sid * E_PER_T
        nbase = sid * SLICE
        cs = pltpu.async_copy(edge_hbm.at[pl.ds(ebase, E_PER_T)], src_v, sem)
        cd = pltpu.async_copy(edge_hbm.at[pl.ds(N_EDGES + ebase, E_PER_T)],
                              dst_v, sem_d)

        zeros = jnp.zeros((L,), jnp.float32)
        ones = jnp.ones((L,), jnp.float32)

        @plsc.parallel_loop(0, N_PAD // L, unroll=8)
        def _(i):
            hist_v[pl.ds(i * L, L)] = zeros

        cd.wait()

        @plsc.parallel_loop(0, E_PER_T // L, unroll=5)
        def _(i):
            idx = dst_v[pl.ds(i * L, L)]
            plsc.addupdate_scatter(hist_v, [idx], ones)

        pltpu.sync_copy(hist_v, shared.at[sid])
        plsc.subcore_barrier()

        for k in range(NS):
            pltpu.sync_copy(shared.at[k, pl.ds(nbase, SLICE)], red_v.at[k])

        def _deg_red(j, carry):
            acc = red_v[0, pl.ds(j * L, L)]
            for k in range(1, NS):
                acc = acc + red_v[k, pl.ds(j * L, L)]
            sl_v[pl.ds(j * L, L)] = _newton_rsqrt(acc + 1.0)
            return carry

        lax.fori_loop(0, VPS, _deg_red, 0)

        pltpu.sync_copy(sl_v, shared2.at[pl.ds(nbase, SLICE)])
        plsc.subcore_barrier()

        cb = pltpu.async_copy(shared2, dinv_v, sem_b)

        @plsc.parallel_loop(0, N_PAD // L, unroll=8)
        def _(i):
            hist_v[pl.ds(i * L, L)] = zeros

        cb.wait()
        cs.wait()

        @plsc.parallel_loop(0, E_PER_T // L, unroll=5)
        def _(i):
            si = src_v[pl.ds(i * L, L)]
            di = dst_v[pl.ds(i * L, L)]
            dvals = plsc.load_gather(dinv_v, [di])
            plsc.addupdate_scatter(hist_v, [si], dvals)

        pltpu.sync_copy(hist_v, shared.at[sid])
        plsc.subcore_barrier()

        for k in range(NS):
            pltpu.sync_copy(shared.at[k, pl.ds(nbase, SLICE)], red_v.at[k])

        def _t_red(j, carry):
            acc = red_v[0, pl.ds(j * L, L)]
            for k in range(1, NS):
                acc = acc + red_v[k, pl.ds(j * L, L)]
            dv = sl_v[pl.ds(j * L, L)]
            sl_v[pl.ds(j * L, L)] = dv * (dv + acc)
            return carry

        lax.fori_loop(0, VPS, _t_red, 0)

        pltpu.sync_copy(sl_v, c_hbm.at[pl.ds(nbase, SLICE)])


@functools.cache
def _build_sc_kernel():
    mesh = plsc.VectorSubcoreMesh(core_axis_name="c", subcore_axis_name="s")
    params = pltpu.CompilerParams(needs_layout_passes=False)
    return pl.kernel(
        _sc_fused_body,
        mesh=mesh,
        out_type=jax.ShapeDtypeStruct((N_PAD,), jnp.float32),
        scratch_types=[
            pltpu.VMEM((E_PER_T,), jnp.int32),
            pltpu.VMEM((E_PER_T,), jnp.int32),
            pltpu.VMEM((N_PAD,), jnp.float32),
            pltpu.VMEM((N_PAD,), jnp.float32),
            pltpu.VMEM((NS, SLICE), jnp.float32),
            pltpu.VMEM((SLICE,), jnp.float32),
            pltpu.VMEM_SHARED((NS, N_PAD), jnp.float32),
            pltpu.VMEM_SHARED((N_PAD,), jnp.float32),
            pltpu.SemaphoreType.DMA,
            pltpu.SemaphoreType.DMA,
            pltpu.SemaphoreType.DMA,
        ],
        compiler_params=params,
    )


def _tc_final(c_ref, x_ref, w_ref, b_ref, wd_ref, bd_ref, out_ref):
    c = c_ref[:, :N_NODES]
    cx = lax.dot_general(c, x_ref[...], (((1,), (0,)), ((), ())),
                         preferred_element_type=jnp.float32)
    pooled = lax.dot_general(cx, w_ref[...], (((1,), (0,)), ((), ())),
                             preferred_element_type=jnp.float32)
    pooled = pooled + float(N_NODES) * b_ref[...]
    logits = lax.dot_general(pooled, wd_ref[...], (((1,), (0,)), ((), ())),
                             preferred_element_type=jnp.float32)
    logits = logits + bd_ref[...]
    m = jnp.max(logits, axis=1, keepdims=True)
    e = jnp.exp(logits - m)
    out_ref[...] = e / jnp.sum(e, axis=1, keepdims=True)


def kernel(x, edge_index, W, b, Wd, bd):
    sc_fused = _build_sc_kernel()
    if edge_index.dtype != jnp.int32:
        edge_index = edge_index.astype(jnp.int32)

    c = sc_fused(edge_index.reshape(2 * N_EDGES))

    out = pl.pallas_call(
        _tc_final,
        out_shape=jax.ShapeDtypeStruct((1, 10), jnp.float32),
    )(c.reshape(1, N_PAD), x, W, b.reshape(1, -1), Wd, bd.reshape(1, -1))
    return out.reshape(10)


# speedup vs baseline: 1.2710x; 1.0267x over previous
import functools

import jax
import jax.numpy as jnp
from jax import lax
from jax.experimental import pallas as pl
from jax.experimental.pallas import tpu as pltpu
from jax.experimental.pallas import tpu_sc as plsc

N_NODES = 10000
N_EDGES = 320000
L = 16
NS = 16
N_PAD = 10240
SLICE = N_PAD // NS
VPS = SLICE // L
E_PER_T = N_EDGES // NS

def _newton_rsqrt(x):
    i = lax.bitcast_convert_type(x, jnp.int32)
    i = jnp.int32(0x5F3759DF) - lax.shift_right_logical(i, 1)
    y = lax.bitcast_convert_type(i, jnp.float32)
    for _ in range(3):
        y = y * (1.5 - 0.5 * x * y * y)
    return y


E_WIN = 20480
E_STEP = 19968


def _sc_fused_body(edge_hbm, c_hbm, echunk_v, hist_v, t_v, dinv_v,
                   red_v, sl_v, shared, shared2, sem):
    cid = lax.axis_index("c")
    sid = lax.axis_index("s")

    @pl.when(cid == 0)
    def _():
        wbase = pl.multiple_of(sid * E_STEP, 512)
        off = sid * 32
        nbase = sid * SLICE
        ce = pltpu.async_copy(edge_hbm.at[:, pl.ds(wbase, E_WIN)], echunk_v,
                              sem)

        zeros = jnp.zeros((L,), jnp.float32)
        ones = jnp.ones((L,), jnp.float32)

        @plsc.parallel_loop(0, N_PAD // L, unroll=8)
        def _(i):
            hist_v[pl.ds(i * L, L)] = zeros
            t_v[pl.ds(i * L, L)] = zeros

        ce.wait()

        @plsc.parallel_loop(0, E_PER_T // L, unroll=10)
        def _(i):
            idx = echunk_v[1, pl.ds(off + i * L, L)]
            plsc.addupdate_scatter(hist_v, [idx], ones)

        pltpu.sync_copy(hist_v, shared.at[sid])
        plsc.subcore_barrier()

        for k in range(NS):
            pltpu.sync_copy(shared.at[k, pl.ds(nbase, SLICE)], red_v.at[k])

        def _deg_red(j, carry):
            acc = red_v[0, pl.ds(j * L, L)]
            for k in range(1, NS):
                acc = acc + red_v[k, pl.ds(j * L, L)]
            sl_v[pl.ds(j * L, L)] = _newton_rsqrt(acc + 1.0)
            return carry

        lax.fori_loop(0, VPS, _deg_red, 0)

        pltpu.sync_copy(sl_v, shared2.at[pl.ds(nbase, SLICE)])
        plsc.subcore_barrier()

        pltpu.sync_copy(shared2, dinv_v)

        @plsc.parallel_loop(0, E_PER_T // L, unroll=10)
        def _(i):
            si = echunk_v[0, pl.ds(off + i * L, L)]
            di = echunk_v[1, pl.ds(off + i * L, L)]
            dvals = plsc.load_gather(dinv_v, [di])
            plsc.addupdate_scatter(t_v, [si], dvals)

        pltpu.sync_copy(t_v, shared.at[sid])
        plsc.subcore_barrier()

        for k in range(NS):
            pltpu.sync_copy(shared.at[k, pl.ds(nbase, SLICE)], red_v.at[k])

        def _t_red(j, carry):
            acc = red_v[0, pl.ds(j * L, L)]
            for k in range(1, NS):
                acc = acc + red_v[k, pl.ds(j * L, L)]
            dv = sl_v[pl.ds(j * L, L)]
            sl_v[pl.ds(j * L, L)] = dv * (dv + acc)
            return carry

        lax.fori_loop(0, VPS, _t_red, 0)

        pltpu.sync_copy(sl_v, c_hbm.at[pl.ds(nbase, SLICE)])


@functools.cache
def _build_sc_kernel():
    mesh = plsc.VectorSubcoreMesh(core_axis_name="c", subcore_axis_name="s")
    params = pltpu.CompilerParams(needs_layout_passes=False)
    return pl.kernel(
        _sc_fused_body,
        mesh=mesh,
        out_type=jax.ShapeDtypeStruct((N_PAD,), jnp.float32),
        scratch_types=[
            pltpu.VMEM((2, E_WIN), jnp.int32),
            pltpu.VMEM((N_PAD,), jnp.float32),
            pltpu.VMEM((N_PAD,), jnp.float32),
            pltpu.VMEM((N_PAD,), jnp.float32),
            pltpu.VMEM((NS, SLICE), jnp.float32),
            pltpu.VMEM((SLICE,), jnp.float32),
            pltpu.VMEM_SHARED((NS, N_PAD), jnp.float32),
            pltpu.VMEM_SHARED((N_PAD,), jnp.float32),
            pltpu.SemaphoreType.DMA,
        ],
        compiler_params=params,
    )


def _tc_final(c_ref, x_ref, w_ref, b_ref, wd_ref, bd_ref, out_ref):
    c = c_ref[:, :N_NODES]
    cx = lax.dot_general(c, x_ref[...], (((1,), (0,)), ((), ())),
                         preferred_element_type=jnp.float32)
    pooled = lax.dot_general(cx, w_ref[...], (((1,), (0,)), ((), ())),
                             preferred_element_type=jnp.float32)
    pooled = pooled + float(N_NODES) * b_ref[...]
    logits = lax.dot_general(pooled, wd_ref[...], (((1,), (0,)), ((), ())),
                             preferred_element_type=jnp.float32)
    logits = logits + bd_ref[...]
    m = jnp.max(logits, axis=1, keepdims=True)
    e = jnp.exp(logits - m)
    out_ref[...] = e / jnp.sum(e, axis=1, keepdims=True)


def kernel(x, edge_index, W, b, Wd, bd):
    sc_fused = _build_sc_kernel()
    if edge_index.dtype != jnp.int32:
        edge_index = edge_index.astype(jnp.int32)

    c = sc_fused(edge_index)

    out = pl.pallas_call(
        _tc_final,
        out_shape=jax.ShapeDtypeStruct((1, 10), jnp.float32),
    )(c.reshape(1, N_PAD), x, W, b.reshape(1, -1), Wd, bd.reshape(1, -1))
    return out.reshape(10)


# speedup vs baseline: 1.4034x; 1.1042x over previous
import functools

import jax
import jax.numpy as jnp
from jax import lax
from jax.experimental import pallas as pl
from jax.experimental.pallas import tpu as pltpu
from jax.experimental.pallas import tpu_sc as plsc

N_NODES = 10000
N_EDGES = 320000
L = 16
NS = 16
N_PAD = 10240
SLICE = N_PAD // NS
VPS = SLICE // L
E_PER_T = N_EDGES // NS

def _newton_rsqrt(x):
    i = lax.bitcast_convert_type(x, jnp.int32)
    i = jnp.int32(0x5F3759DF) - lax.shift_right_logical(i, 1)
    y = lax.bitcast_convert_type(i, jnp.float32)
    for _ in range(3):
        y = y * (1.5 - 0.5 * x * y * y)
    return y


E_WIN = 20480
E_STEP = 19968


def _sc_fused_body(edge_hbm, c_hbm, echunk_v, hist_v, t_v, dinv_v,
                   red_v, sl_v, shared, shared2, sem):
    cid = lax.axis_index("c")
    sid = lax.axis_index("s")

    @pl.when(cid == 0)
    def _():
        wbase = pl.multiple_of(sid * E_STEP, 512)
        off = sid * 32
        nbase = sid * SLICE
        ce = pltpu.async_copy(edge_hbm.at[:, pl.ds(wbase, E_WIN)], echunk_v,
                              sem)

        zeros = jnp.zeros((L,), jnp.float32)
        ones = jnp.ones((L,), jnp.float32)

        @plsc.parallel_loop(0, N_PAD // L, unroll=8)
        def _(i):
            hist_v[pl.ds(i * L, L)] = zeros
            t_v[pl.ds(i * L, L)] = zeros

        ce.wait()

        @plsc.parallel_loop(0, E_PER_T // L, unroll=10)
        def _(i):
            idx = echunk_v[1, pl.ds(off + i * L, L)]
            plsc.addupdate_scatter(hist_v, [idx], ones)

        pltpu.sync_copy(hist_v, shared.at[sid])
        plsc.subcore_barrier()

        pltpu.sync_copy(shared.at[:, pl.ds(nbase, SLICE)], red_v)

        def _deg_red(j, carry):
            acc = red_v[0, pl.ds(j * L, L)]
            for k in range(1, NS):
                acc = acc + red_v[k, pl.ds(j * L, L)]
            sl_v[pl.ds(j * L, L)] = _newton_rsqrt(acc + 1.0)
            return carry

        lax.fori_loop(0, VPS, _deg_red, 0)

        pltpu.sync_copy(sl_v, shared2.at[pl.ds(nbase, SLICE)])
        plsc.subcore_barrier()

        pltpu.sync_copy(shared2, dinv_v)

        @plsc.parallel_loop(0, E_PER_T // L, unroll=10)
        def _(i):
            si = echunk_v[0, pl.ds(off + i * L, L)]
            di = echunk_v[1, pl.ds(off + i * L, L)]
            dvals = plsc.load_gather(dinv_v, [di])
            plsc.addupdate_scatter(t_v, [si], dvals)

        pltpu.sync_copy(t_v, shared.at[sid])
        plsc.subcore_barrier()

        pltpu.sync_copy(shared.at[:, pl.ds(nbase, SLICE)], red_v)

        def _t_red(j, carry):
            acc = red_v[0, pl.ds(j * L, L)]
            for k in range(1, NS):
                acc = acc + red_v[k, pl.ds(j * L, L)]
            dv = sl_v[pl.ds(j * L, L)]
            sl_v[pl.ds(j * L, L)] = dv * (dv + acc)
            return carry

        lax.fori_loop(0, VPS, _t_red, 0)

        @pl.when(sid == NS - 1)
        def _():
            for jj in range((N_NODES - (NS - 1) * SLICE) // L, VPS):
                sl_v[pl.ds(jj * L, L)] = zeros

        pltpu.sync_copy(sl_v, c_hbm.at[pl.ds(nbase, SLICE)])


@functools.cache
def _build_sc_kernel():
    mesh = plsc.VectorSubcoreMesh(core_axis_name="c", subcore_axis_name="s")
    params = pltpu.CompilerParams(needs_layout_passes=False)
    return pl.kernel(
        _sc_fused_body,
        mesh=mesh,
        out_type=jax.ShapeDtypeStruct((N_PAD,), jnp.float32),
        scratch_types=[
            pltpu.VMEM((2, E_WIN), jnp.int32),
            pltpu.VMEM((N_PAD,), jnp.float32),
            pltpu.VMEM((N_PAD,), jnp.float32),
            pltpu.VMEM((N_PAD,), jnp.float32),
            pltpu.VMEM((NS, SLICE), jnp.float32),
            pltpu.VMEM((SLICE,), jnp.float32),
            pltpu.VMEM_SHARED((NS, N_PAD), jnp.float32),
            pltpu.VMEM_SHARED((N_PAD,), jnp.float32),
            pltpu.SemaphoreType.DMA,
        ],
        compiler_params=params,
    )


def _tc_final(c_ref, x_ref, w_ref, b_ref, wd_ref, bd_ref, out_ref):
    c = c_ref[:, :N_NODES]
    cx = lax.dot_general(c, x_ref[...], (((1,), (0,)), ((), ())),
                         preferred_element_type=jnp.float32)
    pooled = lax.dot_general(cx, w_ref[...], (((1,), (0,)), ((), ())),
                             preferred_element_type=jnp.float32)
    pooled = pooled + float(N_NODES) * b_ref[...]
    logits = lax.dot_general(pooled, wd_ref[...], (((1,), (0,)), ((), ())),
                             preferred_element_type=jnp.float32)
    logits = logits + bd_ref[...]
    m = jnp.max(logits, axis=1, keepdims=True)
    e = jnp.exp(logits - m)
    out_ref[...] = e / jnp.sum(e, axis=1, keepdims=True)


def kernel(x, edge_index, W, b, Wd, bd):
    sc_fused = _build_sc_kernel()
    if edge_index.dtype != jnp.int32:
        edge_index = edge_index.astype(jnp.int32)

    c = sc_fused(edge_index)

    out = pl.pallas_call(
        _tc_final,
        out_shape=jax.ShapeDtypeStruct((1, 10), jnp.float32),
    )(c.reshape(1, N_PAD), x, W, b.reshape(1, -1), Wd, bd.reshape(1, -1))
    return out.reshape(10)


# speedup vs baseline: 1.4605x; 1.0407x over previous
import functools

import jax
import jax.numpy as jnp
from jax import lax
from jax.experimental import pallas as pl
from jax.experimental.pallas import tpu as pltpu
from jax.experimental.pallas import tpu_sc as plsc

N_NODES = 10000
N_EDGES = 320000
L = 16
NS = 16
N_PAD = 10240
SLICE = N_PAD // NS
VPS = SLICE // L
E_PER_T = N_EDGES // NS

def _newton_rsqrt(x):
    i = lax.bitcast_convert_type(x, jnp.int32)
    i = jnp.int32(0x5F3759DF) - lax.shift_right_logical(i, 1)
    y = lax.bitcast_convert_type(i, jnp.float32)
    for _ in range(3):
        y = y * (1.5 - 0.5 * x * y * y)
    return y


E_WIN = 20480
E_STEP = 19968


def _sc_fused_body(edge_hbm, tpart_hbm, dinv_hbm, echunk_v, hist_v, t_v,
                   dinv_v, red_v, sl_v, shared, shared2, sem):
    cid = lax.axis_index("c")
    sid = lax.axis_index("s")

    @pl.when(cid == 0)
    def _():
        wbase = pl.multiple_of(sid * E_STEP, 512)
        off = sid * 32
        nbase = sid * SLICE
        ce = pltpu.async_copy(edge_hbm.at[:, pl.ds(wbase, E_WIN)], echunk_v,
                              sem)

        zeros = jnp.zeros((L,), jnp.float32)
        ones = jnp.ones((L,), jnp.float32)

        @plsc.parallel_loop(0, N_PAD // L, unroll=8)
        def _(i):
            hist_v[pl.ds(i * L, L)] = zeros
            t_v[pl.ds(i * L, L)] = zeros

        ce.wait()

        @plsc.parallel_loop(0, E_PER_T // L, unroll=10)
        def _(i):
            idx = echunk_v[1, pl.ds(off + i * L, L)]
            plsc.addupdate_scatter(hist_v, [idx], ones)

        pltpu.sync_copy(hist_v, shared.at[sid])
        plsc.subcore_barrier()

        pltpu.sync_copy(shared.at[:, pl.ds(nbase, SLICE)], red_v)

        def _deg_red(j, carry):
            acc = red_v[0, pl.ds(j * L, L)]
            for k in range(1, NS):
                acc = acc + red_v[k, pl.ds(j * L, L)]
            sl_v[pl.ds(j * L, L)] = _newton_rsqrt(acc + 1.0)
            return carry

        lax.fori_loop(0, VPS, _deg_red, 0)

        @pl.when(sid == NS - 1)
        def _():
            for jj in range((N_NODES - (NS - 1) * SLICE) // L, VPS):
                sl_v[pl.ds(jj * L, L)] = zeros

        pltpu.sync_copy(sl_v, shared2.at[pl.ds(nbase, SLICE)])
        pltpu.sync_copy(sl_v, dinv_hbm.at[pl.ds(nbase, SLICE)])
        plsc.subcore_barrier()

        pltpu.sync_copy(shared2, dinv_v)

        @plsc.parallel_loop(0, E_PER_T // L, unroll=10)
        def _(i):
            si = echunk_v[0, pl.ds(off + i * L, L)]
            di = echunk_v[1, pl.ds(off + i * L, L)]
            dvals = plsc.load_gather(dinv_v, [di])
            plsc.addupdate_scatter(t_v, [si], dvals)

        pltpu.sync_copy(t_v, tpart_hbm.at[sid])


@functools.cache
def _build_sc_kernel():
    mesh = plsc.VectorSubcoreMesh(core_axis_name="c", subcore_axis_name="s")
    params = pltpu.CompilerParams(needs_layout_passes=False)
    return pl.kernel(
        _sc_fused_body,
        mesh=mesh,
        out_type=(
            jax.ShapeDtypeStruct((NS, N_PAD), jnp.float32),
            jax.ShapeDtypeStruct((N_PAD,), jnp.float32),
        ),
        scratch_types=[
            pltpu.VMEM((2, E_WIN), jnp.int32),
            pltpu.VMEM((N_PAD,), jnp.float32),
            pltpu.VMEM((N_PAD,), jnp.float32),
            pltpu.VMEM((N_PAD,), jnp.float32),
            pltpu.VMEM((NS, SLICE), jnp.float32),
            pltpu.VMEM((SLICE,), jnp.float32),
            pltpu.VMEM_SHARED((NS, N_PAD), jnp.float32),
            pltpu.VMEM_SHARED((N_PAD,), jnp.float32),
            pltpu.SemaphoreType.DMA,
        ],
        compiler_params=params,
    )


def _tc_final(tpart_ref, dinv_ref, x_ref, w_ref, b_ref, wd_ref, bd_ref,
              out_ref):
    t = jnp.sum(tpart_ref[...], axis=0, keepdims=True)
    dinv = dinv_ref[...]
    c = (dinv * (dinv + t))[:, :N_NODES]
    cx = lax.dot_general(c, x_ref[...], (((1,), (0,)), ((), ())),
                         preferred_element_type=jnp.float32)
    pooled = lax.dot_general(cx, w_ref[...], (((1,), (0,)), ((), ())),
                             preferred_element_type=jnp.float32)
    pooled = pooled + float(N_NODES) * b_ref[...]
    logits = lax.dot_general(pooled, wd_ref[...], (((1,), (0,)), ((), ())),
                             preferred_element_type=jnp.float32)
    logits = logits + bd_ref[...]
    m = jnp.max(logits, axis=1, keepdims=True)
    e = jnp.exp(logits - m)
    out_ref[...] = e / jnp.sum(e, axis=1, keepdims=True)


def kernel(x, edge_index, W, b, Wd, bd):
    sc_fused = _build_sc_kernel()
    if edge_index.dtype != jnp.int32:
        edge_index = edge_index.astype(jnp.int32)

    t_part, dinv = sc_fused(edge_index)

    out = pl.pallas_call(
        _tc_final,
        out_shape=jax.ShapeDtypeStruct((1, 10), jnp.float32),
    )(t_part, dinv.reshape(1, N_PAD), x, W, b.reshape(1, -1), Wd,
      bd.reshape(1, -1))
    return out.reshape(10)


# speedup vs baseline: 1.4942x; 1.0230x over previous
import functools

import jax
import jax.numpy as jnp
from jax import lax
from jax.experimental import pallas as pl
from jax.experimental.pallas import tpu as pltpu
from jax.experimental.pallas import tpu_sc as plsc

N_NODES = 10000
N_EDGES = 320000
L = 16
NS = 16
N_PAD = 10240
SLICE = N_PAD // NS
VPS = SLICE // L
E_PER_T = N_EDGES // NS

def _newton_rsqrt(x):
    i = lax.bitcast_convert_type(x, jnp.int32)
    i = jnp.int32(0x5F3759DF) - lax.shift_right_logical(i, 1)
    y = lax.bitcast_convert_type(i, jnp.float32)
    for _ in range(3):
        y = y * (1.5 - 0.5 * x * y * y)
    return y


E_WIN = 20480
E_STEP = 19968


def _sc_fused_body(edge_hbm, tpart_hbm, dinv_hbm, echunk_v, hist_v, t_v,
                   dinv_v, red_v, sl_v, shared, shared2, sem):
    cid = lax.axis_index("c")
    sid = lax.axis_index("s")

    @pl.when(cid == 0)
    def _():
        wbase = pl.multiple_of(sid * E_STEP, 512)
        off = sid * 32
        nbase = sid * SLICE
        ce = pltpu.async_copy(edge_hbm.at[:, pl.ds(wbase, E_WIN)], echunk_v,
                              sem)

        zeros = jnp.zeros((L,), jnp.float32)
        ones = jnp.ones((L,), jnp.float32)

        @plsc.parallel_loop(0, N_PAD // L, unroll=16)
        def _(i):
            hist_v[pl.ds(i * L, L)] = zeros
            t_v[pl.ds(i * L, L)] = zeros

        ce.wait()

        @plsc.parallel_loop(0, E_PER_T // L, unroll=10)
        def _(i):
            idx = echunk_v[1, pl.ds(off + i * L, L)]
            plsc.addupdate_scatter(hist_v, [idx], ones)

        pltpu.sync_copy(hist_v, shared.at[sid])
        plsc.subcore_barrier()

        pltpu.sync_copy(shared.at[:, pl.ds(nbase, SLICE)], red_v)

        @plsc.parallel_loop(0, VPS, unroll=4)
        def _(j):
            acc = red_v[0, pl.ds(j * L, L)]
            for k in range(1, NS):
                acc = acc + red_v[k, pl.ds(j * L, L)]
            sl_v[pl.ds(j * L, L)] = _newton_rsqrt(acc + 1.0)

        @pl.when(sid == NS - 1)
        def _():
            for jj in range((N_NODES - (NS - 1) * SLICE) // L, VPS):
                sl_v[pl.ds(jj * L, L)] = zeros

        pltpu.sync_copy(sl_v, shared2.at[pl.ds(nbase, SLICE)])
        pltpu.sync_copy(sl_v, dinv_hbm.at[pl.ds(nbase, SLICE)])
        plsc.subcore_barrier()

        pltpu.sync_copy(shared2, dinv_v)

        @plsc.parallel_loop(0, E_PER_T // L, unroll=10)
        def _(i):
            si = echunk_v[0, pl.ds(off + i * L, L)]
            di = echunk_v[1, pl.ds(off + i * L, L)]
            dvals = plsc.load_gather(dinv_v, [di])
            plsc.addupdate_scatter(t_v, [si], dvals)

        pltpu.sync_copy(t_v, tpart_hbm.at[sid])


@functools.cache
def _build_sc_kernel():
    mesh = plsc.VectorSubcoreMesh(core_axis_name="c", subcore_axis_name="s")
    params = pltpu.CompilerParams(needs_layout_passes=False)
    return pl.kernel(
        _sc_fused_body,
        mesh=mesh,
        out_type=(
            jax.ShapeDtypeStruct((NS, N_PAD), jnp.float32),
            jax.ShapeDtypeStruct((N_PAD,), jnp.float32),
        ),
        scratch_types=[
            pltpu.VMEM((2, E_WIN), jnp.int32),
            pltpu.VMEM((N_PAD,), jnp.float32),
            pltpu.VMEM((N_PAD,), jnp.float32),
            pltpu.VMEM((N_PAD,), jnp.float32),
            pltpu.VMEM((NS, SLICE), jnp.float32),
            pltpu.VMEM((SLICE,), jnp.float32),
            pltpu.VMEM_SHARED((NS, N_PAD), jnp.float32),
            pltpu.VMEM_SHARED((N_PAD,), jnp.float32),
            pltpu.SemaphoreType.DMA,
        ],
        compiler_params=params,
    )


def _tc_final(tpart_ref, dinv_ref, x_ref, w_ref, b_ref, wd_ref, bd_ref,
              out_ref):
    t = jnp.sum(tpart_ref[...], axis=0, keepdims=True)
    dinv = dinv_ref[...]
    c = (dinv * (dinv + t))[:, :N_NODES]
    cx = lax.dot_general(c, x_ref[...], (((1,), (0,)), ((), ())),
                         preferred_element_type=jnp.float32)
    pooled = lax.dot_general(cx, w_ref[...], (((1,), (0,)), ((), ())),
                             preferred_element_type=jnp.float32)
    pooled = pooled + float(N_NODES) * b_ref[...]
    logits = lax.dot_general(pooled, wd_ref[...], (((1,), (0,)), ((), ())),
                             preferred_element_type=jnp.float32)
    logits = logits + bd_ref[...]
    m = jnp.max(logits, axis=1, keepdims=True)
    e = jnp.exp(logits - m)
    out_ref[...] = e / jnp.sum(e, axis=1, keepdims=True)


def kernel(x, edge_index, W, b, Wd, bd):
    sc_fused = _build_sc_kernel()
    if edge_index.dtype != jnp.int32:
        edge_index = edge_index.astype(jnp.int32)

    t_part, dinv = sc_fused(edge_index)

    out = pl.pallas_call(
        _tc_final,
        out_shape=jax.ShapeDtypeStruct((1, 10), jnp.float32),
    )(t_part, dinv.reshape(1, N_PAD), x, W, b.reshape(1, -1), Wd,
      bd.reshape(1, -1))
    return out.reshape(10)
